# baseline (speedup 1.0000x reference)
import jax
import jax.numpy as jnp
from jax import lax
from jax.experimental import pallas as pl

B = 256
CW_DIM = 2048
ED = 64
K = 1024
DC = CW_DIM // ED


def _vq_step(xT_ref, cb_ref, x2_ref, c2_ref, oh_ref, cwT_ref):
    x = xT_ref[0]
    cb = cb_ref[0]
    x2 = x2_ref[0]
    c2 = c2_ref[0]
    xc = lax.dot_general(x, cb, (((1,), (1,)), ((), ())),
                         preferred_element_type=jnp.float32)
    dist = x2 - 2.0 * xc + c2
    m = jnp.min(dist, axis=1, keepdims=True)
    iota = lax.broadcasted_iota(jnp.int32, (B, K), 1)
    idx = jnp.min(jnp.where(dist == m, iota, K), axis=1, keepdims=True)
    oh = (iota == idx).astype(jnp.float32)
    oh_ref[...] = oh
    cwe = lax.dot_general(oh, cb, (((1,), (0,)), ((), ())),
                          preferred_element_type=jnp.float32)
    cwT_ref[0] = x + (cwe - x)


def kernel(cw_q, codebook):
    xT = jnp.swapaxes(cw_q.reshape(B, DC, ED), 0, 1)
    x2 = jnp.sum(xT * xT, axis=-1, keepdims=True)
    c2 = jnp.sum(codebook * codebook, axis=-1)[:, None, :]

    oh_flat, cwT = pl.pallas_call(
        _vq_step,
        grid=(DC,),
        in_specs=[
            pl.BlockSpec((1, B, ED), lambda d: (d, 0, 0)),
            pl.BlockSpec((1, K, ED), lambda d: (d, 0, 0)),
            pl.BlockSpec((1, B, 1), lambda d: (d, 0, 0)),
            pl.BlockSpec((1, 1, K), lambda d: (d, 0, 0)),
        ],
        out_specs=[
            pl.BlockSpec((B, K), lambda d: (0, d)),
            pl.BlockSpec((1, B, ED), lambda d: (d, 0, 0)),
        ],
        out_shape=[
            jax.ShapeDtypeStruct((B, DC * K), jnp.float32),
            jax.ShapeDtypeStruct((DC, B, ED), jnp.float32),
        ],
    )(xT, codebook, x2, c2)

    cw = jnp.swapaxes(cwT, 0, 1).reshape(B, CW_DIM)
    one_hot = oh_flat.reshape(B, DC, K)
    return (cw, one_hot)
